# baseline (speedup 1.0000x reference)
import functools
import jax
import jax.numpy as jnp
import numpy as np
from jax.experimental import pallas as pl

_HI = jax.lax.Precision.HIGHEST

_E_BLOCK = 2000


def _k1_body(oh_ref, rowE_ref, tailE_ref, w_ref, qT_ref, kT_ref, vT_ref,
             hm_ref, expAtt_ref, v_ref):
    rel = jnp.dot(oh_ref[...], w_ref[...], preferred_element_type=jnp.float32, precision=_HI)
    nre = tailE_ref[...] * rel

    def _mm(a, b):
        return jnp.dot(a.astype(jnp.bfloat16), b.astype(jnp.bfloat16),
                       preferred_element_type=jnp.float32)

    q = _mm(rowE_ref[...], qT_ref[...])
    k = _mm(nre, kT_ref[...])
    v = _mm(nre, vT_ref[...])
    att = jnp.dot(q * k, hm_ref[...], preferred_element_type=jnp.float32, precision=_HI)
    att = jnp.clip(att, -10.0, 10.0)
    expAtt_ref[...] = jnp.exp(att)
    v_ref[...] = v


def _k2_body(expAtt_ref, normG_ref, v_ref, he_ref, res_ref):
    att = expAtt_ref[...] / (normG_ref[...] + 1e-8)
    res_ref[...] = jnp.dot(att, he_ref[...],
                           preferred_element_type=jnp.float32, precision=_HI) * v_ref[...]


def _k3_body(oh_ref, kgH_ref, kgT_ref, w_ref, ones_ref, wout_ref):
    rel = jnp.dot(oh_ref[...], w_ref[...], preferred_element_type=jnp.float32, precision=_HI)
    trel = kgT_ref[...] * rel
    hrel = kgH_ref[...] * rel
    t2 = jnp.sum(trel * trel, axis=1, keepdims=True)
    h2 = jnp.sum(hrel * hrel, axis=1, keepdims=True)
    wout_ref[...] = h2 * t2


def _k4_body(w_ref, mG_ref, tailE_ref, e_ref, num_ref):
    e = jnp.exp(w_ref[...] - mG_ref[...])
    e_ref[...] = e
    num_ref[...] = e * tailE_ref[...]


def _eblk(d):
    return pl.BlockSpec((_E_BLOCK, d), lambda i: (i, 0))


def _full(a, b):
    return pl.BlockSpec((a, b), lambda i: (0, 0))


@jax.jit
def _run(entity_emb, edge_index, edge_type, weight, qTrans, kTrans, vTrans):
    n_entities, latdim = entity_emb.shape
    n_rel = weight.shape[0]
    e_count = edge_type.shape[0]
    grid = (e_count // _E_BLOCK,)
    head = edge_index[0]
    tail = edge_index[1]

    oh = jax.nn.one_hot((edge_type - 1) % n_rel, n_rel, dtype=jnp.float32)
    rowE = entity_emb[head]
    tailE = entity_emb[tail]

    hm = jnp.asarray(np.repeat(np.eye(4, dtype=np.float32), latdim // 4,
                               axis=0))
    he = hm.T
    ones_col = jnp.ones((latdim, 1), jnp.float32)

    expAtt, v = pl.pallas_call(
        _k1_body,
        grid=grid,
        in_specs=[_eblk(n_rel), _eblk(latdim), _eblk(latdim),
                  _full(n_rel, latdim), _full(latdim, latdim),
                  _full(latdim, latdim), _full(latdim, latdim),
                  _full(latdim, 4)],
        out_specs=[_eblk(4), _eblk(latdim)],
        out_shape=[jax.ShapeDtypeStruct((e_count, 4), jnp.float32),
                   jax.ShapeDtypeStruct((e_count, latdim), jnp.float32)],
    )(oh, rowE, tailE, weight, qTrans, kTrans, vTrans, hm)

    attNorm = jax.ops.segment_sum(expAtt, head, num_segments=n_entities)
    normG = attNorm[head]

    res = pl.pallas_call(
        _k2_body,
        grid=grid,
        in_specs=[_eblk(4), _eblk(4), _eblk(latdim), _full(4, latdim)],
        out_specs=_eblk(latdim),
        out_shape=jax.ShapeDtypeStruct((e_count, latdim), jnp.float32),
    )(expAtt, normG, v, he)

    kg = jax.ops.segment_sum(res, head, num_segments=n_entities)

    w = pl.pallas_call(
        _k3_body,
        grid=grid,
        in_specs=[_eblk(n_rel), _eblk(latdim), _eblk(latdim),
                  _full(n_rel, latdim), _full(latdim, 1)],
        out_specs=_eblk(1),
        out_shape=jax.ShapeDtypeStruct((e_count, 1), jnp.float32),
    )(oh, kg[head], kg[tail], weight, ones_col)

    m = jax.ops.segment_max(w[:, 0], head, num_segments=n_entities)
    m = jnp.where(jnp.isfinite(m), m, 0.0)

    e_out, num = pl.pallas_call(
        _k4_body,
        grid=grid,
        in_specs=[_eblk(1), _eblk(1), _eblk(latdim)],
        out_specs=[_eblk(1), _eblk(latdim)],
        out_shape=[jax.ShapeDtypeStruct((e_count, 1), jnp.float32),
                   jax.ShapeDtypeStruct((e_count, latdim), jnp.float32)],
    )(w, m[head][:, None], tailE)

    s = jax.ops.segment_sum(e_out[:, 0], head, num_segments=n_entities)
    numN = jax.ops.segment_sum(num, head, num_segments=n_entities)
    return jnp.where(s[:, None] > 0, numN / jnp.where(s[:, None] > 0,
                                                      s[:, None], 1.0), 0.0)


def kernel(entity_emb, user_emb, edge_index, edge_type, interact_mat, weight,
           qTrans, kTrans, vTrans, layer):
    return _run(entity_emb, edge_index.astype(jnp.int32),
                edge_type.astype(jnp.int32), weight, qTrans, kTrans, vTrans)
